# baseline (speedup 1.0000x reference)
import functools
import jax
import jax.numpy as jnp
from jax import lax
from jax.experimental import pallas as pl
from jax.experimental.pallas import tpu as pltpu
from jax.experimental.pallas import tpu_sc as plsc

B, S, D = 4, 96, 768
Z = 0.3
N = B * S * S
HB = B // 2
NH = N // 2
RB = 2048
GRID = NH // RB
SS = S * S
NV = SS // 16


def _proj_body(w_ref, bias_ref, t_ref, vs_ref, ve_ref, lg_ref, pb_ref):
    t = t_ref[...].astype(jnp.bfloat16)
    w = w_ref[...].astype(jnp.bfloat16)
    x = jnp.dot(t, w, preferred_element_type=jnp.float32) + bias_ref[...]
    xt = x.T[0:8, :]
    lg_ref[...] = xt
    ps = (1.0 / (1.0 + jnp.exp(-xt[0:1, :]))) * (vs_ref[0] >= 0)
    pe = (1.0 / (1.0 + jnp.exp(-xt[1:2, :]))) * (ve_ref[0] >= 0)
    bits = jax.lax.bitcast_convert_type(
        jnp.concatenate([ps, pe], axis=0), jnp.int32)
    pb_ref[...] = jnp.concatenate(
        [bits, jnp.zeros((6, RB), jnp.int32)], axis=0)


def _sc_select(pb_hbm, am_hbm, mk_hbm, bbuf, obuf, ambuf, sbuf):
    wid = lax.axis_index("s") * 2 + lax.axis_index("c")
    r = wid % (2 * HB)
    e = r % 2
    b = r // 2

    pltpu.sync_copy(pb_hbm.at[e, pl.ds(b * SS, SS)], bbuf)
    pltpu.sync_copy(am_hbm.at[b], ambuf)

    def lane_sum(vec):
        sbuf[...] = vec
        v = sbuf[...]
        s = v[0]
        for i in range(1, 16):
            s = s + v[i]
        return s

    one_v = jnp.full((16,), 1, jnp.int32)
    zero_v = jnp.zeros((16,), jnp.int32)

    def amsum(i, acc):
        return acc + ambuf[pl.ds(i * 16, 16)]

    msum = lane_sum(lax.fori_loop(0, S // 16, amsum,
                                  jnp.zeros((16,), jnp.int32)))
    mask_len = msum - 2
    length = (mask_len.astype(jnp.float32) * Z).astype(jnp.int32)
    length = jnp.maximum(length, 5)
    k = jnp.minimum(length, mask_len * mask_len)

    def it(_, carry):
        lo, hi = carry
        mid = (lo + hi) >> 1
        mid_v = jnp.full((16,), mid, jnp.int32)

        def cnt_loop(j, acc):
            ge16 = bbuf[pl.ds(j * 16, 16)] >= mid_v
            return jnp.where(ge16, acc + one_v, acc)

        cnt = lane_sum(lax.fori_loop(0, NV, cnt_loop,
                                     jnp.zeros((16,), jnp.int32), unroll=16))
        ge = cnt >= k
        return jnp.where(ge, mid, lo), jnp.where(ge, hi, mid)

    lo, _ = lax.fori_loop(0, 30, it, (jnp.int32(0), jnp.int32(1 << 30)))
    lo_v = jnp.full((16,), lo, jnp.int32)

    def maskloop(j, carry):
        sl = pl.ds(j * 16, 16)
        obuf[sl] = jnp.where(bbuf[sl] >= lo_v, one_v, zero_v)
        return carry

    lax.fori_loop(0, NV, maskloop, 0, unroll=16)
    pltpu.sync_copy(obuf, mk_hbm.at[wid])


_sc_select_call = functools.partial(
    pl.kernel,
    mesh=plsc.VectorSubcoreMesh(core_axis_name="c", subcore_axis_name="s"),
    out_type=jax.ShapeDtypeStruct((32, SS), jnp.int32),
    scratch_types=[
        pltpu.VMEM((SS,), jnp.int32),
        pltpu.VMEM((SS,), jnp.int32),
        pltpu.VMEM((S,), jnp.int32),
        pltpu.VMEM((16,), jnp.int32),
    ],
)(_sc_select)


def _proj_call(w2, bias, t2h, lab_s_h, lab_e_h):
    return pl.pallas_call(
        _proj_body,
        grid=(GRID,),
        in_specs=[
            pl.BlockSpec((D, 128), lambda i: (0, 0)),
            pl.BlockSpec((1, 128), lambda i: (0, 0)),
            pl.BlockSpec((RB, D), lambda i: (i, 0)),
            pl.BlockSpec((1, 1, RB), lambda i: (i, 0, 0)),
            pl.BlockSpec((1, 1, RB), lambda i: (i, 0, 0)),
        ],
        out_specs=[pl.BlockSpec((8, RB), lambda i: (0, i)),
                   pl.BlockSpec((8, RB), lambda i: (0, i))],
        out_shape=[jax.ShapeDtypeStruct((8, NH), jnp.float32),
                   jax.ShapeDtypeStruct((8, NH), jnp.int32)],
    )(w2, bias, t2h, lab_s_h, lab_e_h)


def kernel(table, attention_mask, table_labels_S, table_labels_E, domain_ids,
           mode, W_S, b_S, W_E, b_E):
    t2 = table.reshape(2, NH, D)
    w2 = jnp.zeros((D, 128), jnp.float32)
    w2 = w2.at[:, 0].set(W_S[0]).at[:, 1].set(W_E[0])
    bias = jnp.zeros((1, 128), jnp.float32)
    bias = bias.at[0, 0].set(b_S[0]).at[0, 1].set(b_E[0])
    lab_s = table_labels_S.reshape(2, GRID, 1, RB)
    lab_e = table_labels_E.reshape(2, GRID, 1, RB)

    lg0, pb0 = _proj_call(w2, bias, t2[0], lab_s[0], lab_e[0])
    mk0 = _sc_select_call(pb0[0:2], attention_mask[0:HB])
    lg1, pb1 = _proj_call(w2, bias, t2[1], lab_s[1], lab_e[1])
    mk1 = _sc_select_call(pb1[0:2], attention_mask[HB:])

    logits_s = jnp.concatenate([lg0[0], lg1[0]]).reshape(B, S, S)
    logits_e = jnp.concatenate([lg0[1], lg1[1]]).reshape(B, S, S)
    mask_s = jnp.concatenate([mk0[0:2 * HB:2], mk1[0:2 * HB:2]])
    mask_e = jnp.concatenate([mk0[1:2 * HB:2], mk1[1:2 * HB:2]])

    return (logits_s, logits_e,
            mask_s.reshape(B, S, S).astype(bool),
            mask_e.reshape(B, S, S).astype(bool))


# speedup vs baseline: 1.8300x; 1.8300x over previous
import functools
import jax
import jax.numpy as jnp
from jax import lax
from jax.experimental import pallas as pl
from jax.experimental.pallas import tpu as pltpu
from jax.experimental.pallas import tpu_sc as plsc

B, S, D = 4, 96, 768
Z = 0.3
N = B * S * S
RB = 2048
GRID = N // RB
SS = S * S
NV = SS // 16


def _proj_body(w_ref, bias_ref, t_ref, vs_ref, ve_ref, lg_ref, pb_ref):
    t = t_ref[...].astype(jnp.bfloat16)
    w = w_ref[...].astype(jnp.bfloat16)
    x = jnp.dot(t, w, preferred_element_type=jnp.float32) + bias_ref[...]
    xt = x.T[0:8, :]
    lg_ref[...] = xt
    ps = (1.0 / (1.0 + jnp.exp(-xt[0:1, :]))) * (vs_ref[0] >= 0)
    pe = (1.0 / (1.0 + jnp.exp(-xt[1:2, :]))) * (ve_ref[0] >= 0)
    bits = jax.lax.bitcast_convert_type(
        jnp.concatenate([ps, pe], axis=0), jnp.int32)
    pb_ref[...] = jnp.concatenate(
        [bits, jnp.zeros((6, RB), jnp.int32)], axis=0)


def _sc_select(pb_hbm, am_hbm, mk_hbm, bbuf, obuf, ambuf, sbuf):
    wid = lax.axis_index("s") * 2 + lax.axis_index("c")
    r = wid % (2 * B)
    e = r % 2
    b = r // 2

    pltpu.sync_copy(pb_hbm.at[e, pl.ds(b * SS, SS)], bbuf)
    pltpu.sync_copy(am_hbm.at[b], ambuf)

    def lane_sum(vec):
        sbuf[...] = vec
        v = sbuf[...]
        s = v[0]
        for i in range(1, 16):
            s = s + v[i]
        return s

    one_v = jnp.full((16,), 1, jnp.int32)
    zero_v = jnp.zeros((16,), jnp.int32)

    def amsum(i, acc):
        return acc + ambuf[pl.ds(i * 16, 16)]

    msum = lane_sum(lax.fori_loop(0, S // 16, amsum,
                                  jnp.zeros((16,), jnp.int32)))
    mask_len = msum - 2
    length = (mask_len.astype(jnp.float32) * Z).astype(jnp.int32)
    length = jnp.maximum(length, 5)
    k = jnp.minimum(length, mask_len * mask_len)

    def it(_, carry):
        lo, hi = carry
        mid = (lo + hi) >> 1
        mid_v = jnp.full((16,), mid, jnp.int32)

        def cnt_loop(j, acc):
            ge16 = bbuf[pl.ds(j * 16, 16)] >= mid_v
            return jnp.where(ge16, acc + one_v, acc)

        cnt = lane_sum(lax.fori_loop(0, NV, cnt_loop,
                                     jnp.zeros((16,), jnp.int32), unroll=16))
        ge = cnt >= k
        return jnp.where(ge, mid, lo), jnp.where(ge, hi, mid)

    lo, _ = lax.fori_loop(0, 30, it, (jnp.int32(0), jnp.int32(1 << 30)))
    lo_v = jnp.full((16,), lo, jnp.int32)

    def maskloop(j, carry):
        sl = pl.ds(j * 16, 16)
        obuf[sl] = jnp.where(bbuf[sl] >= lo_v, one_v, zero_v)
        return carry

    lax.fori_loop(0, NV, maskloop, 0, unroll=16)
    pltpu.sync_copy(obuf, mk_hbm.at[wid])


_sc_select_call = functools.partial(
    pl.kernel,
    mesh=plsc.VectorSubcoreMesh(core_axis_name="c", subcore_axis_name="s"),
    out_type=jax.ShapeDtypeStruct((32, SS), jnp.int32),
    scratch_types=[
        pltpu.VMEM((SS,), jnp.int32),
        pltpu.VMEM((SS,), jnp.int32),
        pltpu.VMEM((S,), jnp.int32),
        pltpu.VMEM((16,), jnp.int32),
    ],
)(_sc_select)


def _proj_call(w2, bias, t2h, lab_s_h, lab_e_h):
    return pl.pallas_call(
        _proj_body,
        grid=(GRID,),
        in_specs=[
            pl.BlockSpec((D, 128), lambda i: (0, 0)),
            pl.BlockSpec((1, 128), lambda i: (0, 0)),
            pl.BlockSpec((RB, D), lambda i: (i, 0)),
            pl.BlockSpec((1, 1, RB), lambda i: (i, 0, 0)),
            pl.BlockSpec((1, 1, RB), lambda i: (i, 0, 0)),
        ],
        out_specs=[pl.BlockSpec((8, RB), lambda i: (0, i)),
                   pl.BlockSpec((8, RB), lambda i: (0, i))],
        out_shape=[jax.ShapeDtypeStruct((8, N), jnp.float32),
                   jax.ShapeDtypeStruct((8, N), jnp.int32)],
    )(w2, bias, t2h, lab_s_h, lab_e_h)


def kernel(table, attention_mask, table_labels_S, table_labels_E, domain_ids,
           mode, W_S, b_S, W_E, b_E):
    t2 = table.reshape(N, D)
    w2 = jnp.zeros((D, 128), jnp.float32)
    w2 = w2.at[:, 0].set(W_S[0]).at[:, 1].set(W_E[0])
    bias = jnp.zeros((1, 128), jnp.float32)
    bias = bias.at[0, 0].set(b_S[0]).at[0, 1].set(b_E[0])
    lab_s = table_labels_S.reshape(GRID, 1, RB)
    lab_e = table_labels_E.reshape(GRID, 1, RB)

    lg, pb = _proj_call(w2, bias, t2, lab_s, lab_e)
    mk = _sc_select_call(pb[0:2], attention_mask)

    return (lg[0].reshape(B, S, S), lg[1].reshape(B, S, S),
            mk[0:2 * B:2].reshape(B, S, S).astype(bool),
            mk[1:2 * B:2].reshape(B, S, S).astype(bool))
